# baseline (speedup 1.0000x reference)
import functools

import jax
import jax.numpy as jnp
from jax import lax
from jax.experimental import pallas as pl
from jax.experimental.pallas import tpu as pltpu
from jax.experimental.pallas import tpu_sc as plsc

_BATCH = 4096
_MAX_LEN = 200
_DIM = 64
_PADW = 128
_NW = 32
_G = _BATCH // _NW
_LANES = 16
_NBUF = 2
_NROUND = _MAX_LEN // _NBUF


@functools.partial(
    pl.kernel,
    mesh=plsc.VectorSubcoreMesh(core_axis_name="c", subcore_axis_name="s"),
    out_type=jax.ShapeDtypeStruct((_MAX_LEN, _DIM, _BATCH), jnp.float32),
    compiler_params=pltpu.CompilerParams(
        use_tc_tiling_on_sc=True, needs_layout_passes=False),
    scratch_types=[
        pltpu.VMEM((_MAX_LEN, _G), jnp.int32),
        pltpu.VMEM((_NBUF, _G, _PADW), jnp.float32),
        pltpu.VMEM((_NBUF, _DIM, _G), jnp.float32),
        pltpu.VMEM((_DIM, _MAX_LEN), jnp.float32),
        pltpu.SemaphoreType.DMA((_NBUF,)),
        pltpu.SemaphoreType.DMA((_NBUF,)),
    ],
)
def _emb_lookup(idx_hbm, tok_hbm, pos_hbm, out_hbm,
                idx_v, rows_v, obuf_v, pos_v, sem_g, sem_o):
    wid = lax.axis_index("s") * 2 + lax.axis_index("c")
    b0 = wid * _G
    pltpu.sync_copy(idx_hbm.at[:, pl.ds(b0, _G)], idx_v)
    pltpu.sync_copy(pos_hbm, pos_v)

    def gather(t, b):
        return pltpu.make_async_copy(
            tok_hbm.at[idx_v.at[t]], rows_v.at[b], sem_g.at[b])

    def out_copy(t, b):
        return pltpu.make_async_copy(
            obuf_v.at[b], out_hbm.at[t, :, pl.ds(b0, _G)], sem_o.at[b])

    for b in range(_NBUF):
        gather(b, b).start()

    rids0 = tuple(lax.iota(jnp.int32, _LANES) + (k * _LANES)
                  for k in range(_G // _LANES))

    def round_body(m, rids):
        for b in range(_NBUF):
            t = m * _NBUF + b
            gather(t, b).wait()

            @pl.when(m >= 1)
            def _():
                out_copy(t - _NBUF, b).wait()

            tvec = jnp.full((_LANES,), t, jnp.int32)

            def drow(d, c):
                dvec = jnp.full((_LANES,), d, jnp.int32)
                pvec = plsc.load_gather(pos_v, [dvec, tvec])
                for k in range(_G // _LANES):
                    g = plsc.load_gather(rows_v.at[b], [c[k], dvec])
                    obuf_v[b, d, pl.ds(k * _LANES, _LANES)] = g + pvec
                return c

            lax.fori_loop(0, _DIM, drow, rids, unroll=2)
            out_copy(t, b).start()

            @pl.when(m < _NROUND - 1)
            def _():
                gather(t + _NBUF, b).start()
        return rids

    lax.fori_loop(0, _NROUND, round_body, rids0)

    for b in range(_NBUF):
        out_copy(_MAX_LEN - _NBUF + b, b).wait()


def kernel(inputs, token_table, pos_table):
    idx_t = inputs.T.astype(jnp.int32)
    tok_pad = jnp.pad(token_table, ((0, 0), (0, _PADW - _DIM)))
    pos_t = pos_table.T
    out3 = _emb_lookup(idx_t, tok_pad, pos_t)
    return out3.transpose(2, 0, 1)


# speedup vs baseline: 1.3123x; 1.3123x over previous
import functools

import jax
import jax.numpy as jnp
from jax import lax
from jax.experimental import pallas as pl
from jax.experimental.pallas import tpu as pltpu
from jax.experimental.pallas import tpu_sc as plsc

_BATCH = 4096
_MAX_LEN = 200
_DIM = 64
_PADW = 128
_NW = 32
_G = _BATCH // _NW
_GP = _G // 2
_LANES = 16
_VPR = _DIM // _LANES
_NBUF = 4
_OBUF = 2
_NROUND = _MAX_LEN // _NBUF


@functools.partial(
    pl.kernel,
    mesh=plsc.VectorSubcoreMesh(core_axis_name="c", subcore_axis_name="s"),
    out_type=jax.ShapeDtypeStruct((_MAX_LEN, _BATCH // 2, _PADW), jnp.float32),
    compiler_params=pltpu.CompilerParams(
        use_tc_tiling_on_sc=True, needs_layout_passes=False),
    scratch_types=[
        pltpu.VMEM((_NBUF, _G), jnp.int32),
        pltpu.VMEM((_NBUF, _G, _PADW), jnp.float32),
        pltpu.VMEM((_OBUF, _GP, _PADW), jnp.float32),
        pltpu.VMEM((_DIM, _MAX_LEN), jnp.float32),
        pltpu.VMEM((_MAX_LEN, _DIM), jnp.float32),
        pltpu.SemaphoreType.DMA((_NBUF,)),
        pltpu.SemaphoreType.DMA((_NBUF,)),
        pltpu.SemaphoreType.DMA((_OBUF,)),
    ],
)
def _emb_lookup(idx_hbm, tok_hbm, pos_hbm, out_hbm,
                idx_v, rows_v, obuf_v, post_v, posr_v, sem_i, sem_g, sem_o):
    wid = lax.axis_index("s") * 2 + lax.axis_index("c")
    b0 = wid * _G
    o0 = wid * _GP
    pltpu.sync_copy(pos_hbm, post_v)

    def pos_t(t, c):
        tvec = jnp.full((_LANES,), t, jnp.int32)
        for k in range(_VPR):
            posr_v[t, pl.ds(k * _LANES, _LANES)] = plsc.load_gather(
                post_v, [lax.iota(jnp.int32, _LANES) + (k * _LANES), tvec])
        return c

    lax.fori_loop(0, _MAX_LEN, pos_t, 0)

    def idx_copy(t, s):
        return pltpu.make_async_copy(
            idx_hbm.at[t, pl.ds(b0, _G)], idx_v.at[s], sem_i.at[s])

    def gather(b):
        return pltpu.make_async_copy(
            tok_hbm.at[idx_v.at[b]], rows_v.at[b], sem_g.at[b])

    def out_copy(t, b):
        return pltpu.make_async_copy(
            obuf_v.at[b], out_hbm.at[t, pl.ds(o0, _GP)], sem_o.at[b])

    for b in range(_NBUF - 1):
        idx_copy(b, b).start()
    for b in range(_NBUF - 1):
        idx_copy(b, b).wait()
        gather(b).start()
    idx_copy(_NBUF - 1, _NBUF - 1).start()

    def round_body(m, carry):
        for b in range(_NBUF):
            t = m * _NBUF + b
            gather(b).wait()
            ob = b % _OBUF
            if b >= _OBUF:
                out_copy(t - _OBUF, ob).wait()
            else:
                @pl.when(m >= 1)
                def _():
                    out_copy(t - _OBUF, ob).wait()

            pvecs = [posr_v[t, pl.ds(k * _LANES, _LANES)]
                     for k in range(_VPR)]

            def pair(rp, c):
                for half in range(2):
                    r = rp * 2 + half
                    for k in range(_VPR):
                        obuf_v[ob, rp, pl.ds(half * _DIM + k * _LANES, _LANES)] = (
                            rows_v[b, r, pl.ds(k * _LANES, _LANES)] + pvecs[k])
                return c

            lax.fori_loop(0, _GP, pair, 0, unroll=4)
            out_copy(t, ob).start()

            b2 = (b + _NBUF - 1) % _NBUF

            @pl.when(t + _NBUF - 1 < _MAX_LEN)
            def _():
                idx_copy(t + _NBUF - 1, b2).wait()
                gather(b2).start()

            @pl.when(t + _NBUF < _MAX_LEN)
            def _():
                idx_copy(t + _NBUF, b).start()
        return carry

    lax.fori_loop(0, _NROUND, round_body, 0)
    for b in range(_OBUF):
        out_copy(_MAX_LEN - _OBUF + b, b).wait()


def kernel(inputs, token_table, pos_table):
    idx_t = inputs.T.astype(jnp.int32)
    tok_pad = jnp.pad(token_table, ((0, 0), (0, _PADW - _DIM)))
    pos_t = pos_table.T
    packed = _emb_lookup(idx_t, tok_pad, pos_t)
    return packed.reshape(_MAX_LEN, _BATCH, _DIM).transpose(1, 0, 2)


# speedup vs baseline: 1.5240x; 1.1613x over previous
import functools

import jax
import jax.numpy as jnp
from jax import lax
from jax.experimental import pallas as pl
from jax.experimental.pallas import tpu as pltpu
from jax.experimental.pallas import tpu_sc as plsc

_BATCH = 4096
_MAX_LEN = 200
_DIM = 64
_N = _BATCH * _MAX_LEN
_NW = 32
_RPW = _N // _NW
_NSEQ = _RPW // _MAX_LEN
_NBUF = 4
_G0 = 128
_G1 = _MAX_LEN - _G0
_LANES = 16
_VPR = _DIM // _LANES
_NROUND = _NSEQ // _NBUF


@functools.partial(
    pl.kernel,
    mesh=plsc.VectorSubcoreMesh(core_axis_name="c", subcore_axis_name="s"),
    out_type=jax.ShapeDtypeStruct((_N, _DIM), jnp.float32),
    compiler_params=pltpu.CompilerParams(use_tc_tiling_on_sc=False),
    scratch_types=[
        pltpu.VMEM((_RPW,), jnp.int32),
        pltpu.VMEM((_NBUF, _MAX_LEN, _DIM), jnp.float32),
        pltpu.VMEM((_MAX_LEN, _DIM), jnp.float32),
        pltpu.SemaphoreType.DMA((_NBUF,)),
        pltpu.SemaphoreType.DMA((_NBUF,)),
    ],
)
def _emb_lookup(idx_hbm, tok_hbm, pos_hbm, out_hbm, idx_v, rows_v, pos_v, sem_g, sem_o):
    wid = lax.axis_index("s") * 2 + lax.axis_index("c")
    base_w = wid * _RPW
    pltpu.sync_copy(idx_hbm.at[pl.ds(base_w, _RPW)], idx_v)
    pltpu.sync_copy(pos_hbm, pos_v)

    def fire_gather(seq, b):
        off = seq * _MAX_LEN
        pltpu.make_async_copy(
            tok_hbm.at[idx_v.at[pl.ds(off, _G0)]],
            rows_v.at[b, pl.ds(0, _G0)], sem_g.at[b]).start()
        pltpu.make_async_copy(
            tok_hbm.at[idx_v.at[pl.ds(off + _G0, _G1)]],
            rows_v.at[b, pl.ds(_G0, _G1)], sem_g.at[b]).start()

    def wait_gather(seq, b):
        off = seq * _MAX_LEN
        pltpu.make_async_copy(
            tok_hbm.at[idx_v.at[pl.ds(off, _G0)]],
            rows_v.at[b, pl.ds(0, _G0)], sem_g.at[b]).wait()
        pltpu.make_async_copy(
            tok_hbm.at[idx_v.at[pl.ds(off + _G0, _G1)]],
            rows_v.at[b, pl.ds(_G0, _G1)], sem_g.at[b]).wait()

    def out_copy(seq, b):
        return pltpu.make_async_copy(
            rows_v.at[b], out_hbm.at[pl.ds(base_w + seq * _MAX_LEN, _MAX_LEN)],
            sem_o.at[b])

    for b in range(_NBUF - 1):
        fire_gather(b, b)

    def round_body(m, carry):
        for b in range(_NBUF):
            j = m * _NBUF + b
            wait_gather(j, b)
            def row(r, c):
                for k in range(_VPR):
                    plsc.addupdate(rows_v.at[b, r, pl.ds(k * _LANES, _LANES)],
                                   pos_v[r, pl.ds(k * _LANES, _LANES)])
                return c
            lax.fori_loop(0, _MAX_LEN, row, 0, unroll=4)
            out_copy(j, b).start()
            b2 = (b - 1) % _NBUF
            j2 = j + _NBUF - 1

            @pl.when(j >= 1)
            def _():
                out_copy(j - 1, b2).wait()

            @pl.when(j2 < _NSEQ)
            def _():
                fire_gather(j2, b2)
        return carry

    lax.fori_loop(0, _NROUND, round_body, 0)

    out_copy(_NSEQ - 1, (_NSEQ - 1) % _NBUF).wait()


def kernel(inputs, token_table, pos_table):
    idx = inputs.reshape(-1).astype(jnp.int32)
    out = _emb_lookup(idx, token_table, pos_table)
    return out.reshape(_BATCH, _MAX_LEN, _DIM)
